# Initial kernel scaffold; baseline (speedup 1.0000x reference)
import jax
import jax.numpy as jnp
from jax.experimental import pallas as pl


def kernel(x, edge_index, batch, W11, b11, W12, b12, W21, b21, W22, b22, Wfc, bfc):
    raise NotImplementedError("write your pallas kernel here")


# speedup vs baseline: 1.0004x; 1.0004x over previous
import jax
import jax.numpy as jnp
from jax.experimental import pallas as pl


def _logsoftmax_kernel(x_ref, o_ref):
    x = x_ref[...]
    m = jnp.max(x, axis=1, keepdims=True)
    s = jnp.log(jnp.sum(jnp.exp(x - m), axis=1, keepdims=True))
    o_ref[...] = x - m - s


def _make_norm(row, col, n, dtype):
    deg = jax.ops.segment_sum(jnp.ones_like(row, dtype=dtype), row, num_segments=n)
    dis = jnp.where(deg > 0, 1.0 / jnp.sqrt(deg), 0.0)
    return -dis[row] * dis[col]


def kernel(x, edge_index, batch, W11, b11, W12, b12, W21, b21, W22, b22, Wfc, bfc):
    n = x.shape[0]
    G = 16
    src = edge_index[0]
    dst = edge_index[1]
    norm_f = _make_norm(src, dst, n, x.dtype)
    norm_r = _make_norm(dst, src, n, x.dtype)

    def cheb(xin, s, d, nrm, W, b):
        def matvec(t):
            return jax.ops.segment_sum(nrm[:, None] * t[s], d, num_segments=n)
        Tx0 = xin
        out = Tx0 @ W[0]
        Tx1 = matvec(Tx0)
        out = out + Tx1 @ W[1]
        for k in range(2, W.shape[0]):
            Tx2 = 2.0 * matvec(Tx1) - Tx0
            out = out + Tx2 @ W[k]
            Tx0, Tx1 = Tx1, Tx2
        return out + b

    x1 = jax.nn.relu(cheb(x, src, dst, norm_f, W11, b11))
    x2 = jax.nn.relu(cheb(x, dst, src, norm_r, W12, b12))
    h = jnp.concatenate([x1, x2], axis=1)
    x1 = jax.nn.relu(cheb(h, src, dst, norm_f, W21, b21))
    x2 = jax.nn.relu(cheb(h, dst, src, norm_r, W22, b22))
    h = jnp.concatenate([x1, x2], axis=1)
    sums = jax.ops.segment_sum(h, batch, num_segments=G)
    cnt = jax.ops.segment_sum(jnp.ones((n,), dtype=h.dtype), batch, num_segments=G)
    pooled = sums / jnp.maximum(cnt, 1.0)[:, None]
    logits = pooled @ Wfc + bfc
    return pl.pallas_call(
        _logsoftmax_kernel,
        out_shape=jax.ShapeDtypeStruct(logits.shape, logits.dtype),
    )(logits)


# speedup vs baseline: 5.2785x; 5.2763x over previous
import functools

import jax
import jax.numpy as jnp
from jax import lax
from jax.experimental import pallas as pl
from jax.experimental.pallas import tpu as pltpu
from jax.experimental.pallas import tpu_sc as plsc

N = 10000
NPAD = 10240
F = 128
G = 16
NTILES = 16
CHUNK = 128
ROWS_PER_TILE = NPAD // NTILES
BLK = 1024
NBLK = NPAD // BLK


def _sc_matvec(zf, zr, srcp, dstp):
    epad = srcp.shape[0]
    per_tile = epad // NTILES
    nchunks = per_tile // CHUNK
    mesh = plsc.VectorSubcoreMesh(core_axis_name="c", subcore_axis_name="s")

    @functools.partial(
        pl.kernel,
        out_type=[jax.ShapeDtypeStruct((NPAD, F), jnp.float32)] * 2,
        mesh=mesh,
        scratch_types=[
            pltpu.VMEM_SHARED((NPAD, F), jnp.float32),
            pltpu.VMEM((CHUNK,), jnp.int32),
            pltpu.VMEM((1, CHUNK), jnp.int32),
            pltpu.VMEM((CHUNK, F), jnp.float32),
            pltpu.VMEM((CHUNK, F), jnp.float32),
        ],
    )
    def k(zf_hbm, zr_hbm, src_hbm, dst_hbm, yf_hbm, yr_hbm,
          acc, gidx, sidx, rows, zbuf):
        s = lax.axis_index("s")

        def run(z_hbm, g_hbm, sc_hbm, y_hbm):
            @pl.loop(0, CHUNK)
            def _(r):
                @pl.loop(0, F // 16)
                def _(j):
                    zbuf[r, pl.ds(j * 16, 16)] = jnp.zeros((16,), jnp.float32)

            @pl.loop(0, ROWS_PER_TILE // CHUNK)
            def _(t):
                pltpu.sync_copy(zbuf, acc.at[pl.ds(s * ROWS_PER_TILE + t * CHUNK, CHUNK)])

            plsc.subcore_barrier()

            @pl.loop(0, nchunks)
            def _(i):
                base = s * per_tile + i * CHUNK
                pltpu.sync_copy(g_hbm.at[pl.ds(base, CHUNK)], gidx)
                pltpu.sync_copy(sc_hbm.at[pl.ds(base, CHUNK)], sidx.at[0])
                pltpu.sync_copy(z_hbm.at[gidx], rows)
                pltpu.sync_copy(rows, acc.at[sidx.at[0]], add=True)

            plsc.subcore_barrier()
            pltpu.sync_copy(acc.at[pl.ds(s * ROWS_PER_TILE, ROWS_PER_TILE)],
                            y_hbm.at[pl.ds(s * ROWS_PER_TILE, ROWS_PER_TILE)])

        c = lax.axis_index("c")

        @pl.when(c == 0)
        def _():
            run(zf_hbm, src_hbm, dst_hbm, yf_hbm)

        @pl.when(c == 1)
        def _():
            run(zr_hbm, dst_hbm, src_hbm, yr_hbm)

    return k(zf, zr, srcp, dstp)


def _dis(deg_blk):
    d = deg_blk[:, 0:1]
    return jnp.where(d > 0.0, lax.rsqrt(d), 0.0)


def _feat_spec():
    return pl.BlockSpec((BLK, F), lambda i: (i, 0))


def _deg_spec():
    return pl.BlockSpec((BLK, F), lambda i: (i, 0))


def _tc_prep(xp, degf, degr):
    def body(x_ref, df_ref, dr_ref, zf_ref, zr_ref):
        x = x_ref[...]
        zf_ref[...] = _dis(df_ref[...]) * x
        zr_ref[...] = _dis(dr_ref[...]) * x

    return pl.pallas_call(
        body,
        grid=(NBLK,),
        in_specs=[_feat_spec(), _deg_spec(), _deg_spec()],
        out_specs=[_feat_spec(), _feat_spec()],
        out_shape=[jax.ShapeDtypeStruct((NPAD, F), jnp.float32)] * 2,
    )(xp, degf, degr)


def _tc_step(sf, sr, prevf, prevr, degf, degr, a):
    has_prev = prevf is not None

    def body(*refs):
        if has_prev:
            sf_ref, sr_ref, pf_ref, pr_ref, df_ref, dr_ref, tf_ref, tr_ref, zf_ref, zr_ref = refs
        else:
            sf_ref, sr_ref, df_ref, dr_ref, tf_ref, tr_ref, zf_ref, zr_ref = refs
        disf = _dis(df_ref[...])
        disr = _dis(dr_ref[...])
        txf = -a * disf * sf_ref[...]
        txr = -a * disr * sr_ref[...]
        if has_prev:
            txf = txf - pf_ref[...]
            txr = txr - pr_ref[...]
        tf_ref[...] = txf
        tr_ref[...] = txr
        zf_ref[...] = disf * txf
        zr_ref[...] = disr * txr

    nin = 2 + (2 if has_prev else 0)
    args = (sf, sr) + ((prevf, prevr) if has_prev else ()) + (degf, degr)
    return pl.pallas_call(
        body,
        grid=(NBLK,),
        in_specs=[_feat_spec()] * nin + [_deg_spec(), _deg_spec()],
        out_specs=[_feat_spec()] * 4,
        out_shape=[jax.ShapeDtypeStruct((NPAD, F), jnp.float32)] * 4,
    )(*args)


def _tc_layer_end(tx_f, tx_r, Wf, Wr, bf, br, degf, degr, emit_z):
    H = Wf.shape[1]

    def body(*refs):
        (t0f, t1f, t2f, t3f, t0r, t1r, t2r, t3r,
         wf_ref, wr_ref, bf_ref, br_ref) = refs[:12]
        rest = refs[12:]
        catf = jnp.concatenate([t0f[...], t1f[...], t2f[...], t3f[...]], axis=1)
        catr = jnp.concatenate([t0r[...], t1r[...], t2r[...], t3r[...]], axis=1)
        of = jnp.maximum(
            jnp.dot(catf, wf_ref[...], preferred_element_type=jnp.float32)
            + bf_ref[...], 0.0)
        orr = jnp.maximum(
            jnp.dot(catr, wr_ref[...], preferred_element_type=jnp.float32)
            + br_ref[...], 0.0)
        h = jnp.concatenate([of, orr], axis=1)
        if emit_z:
            df_ref, dr_ref, h_ref, zf_ref, zr_ref = rest
            h_ref[...] = h
            zf_ref[...] = _dis(df_ref[...]) * h
            zr_ref[...] = _dis(dr_ref[...]) * h
        else:
            (h_ref,) = rest
            h_ref[...] = h

    w_spec = pl.BlockSpec((4 * F, H), lambda i: (0, 0))
    b_spec = pl.BlockSpec((1, H), lambda i: (0, 0))
    h_spec = pl.BlockSpec((BLK, 2 * H), lambda i: (i, 0))
    in_specs = [_feat_spec()] * 8 + [w_spec, w_spec, b_spec, b_spec]
    args = tuple(tx_f) + tuple(tx_r) + (Wf, Wr, bf, br)
    if emit_z:
        in_specs += [_deg_spec(), _deg_spec()]
        args += (degf, degr)
        out_specs = [h_spec, h_spec, h_spec]
        out_shape = [jax.ShapeDtypeStruct((NPAD, 2 * H), jnp.float32)] * 3
    else:
        out_specs = [h_spec]
        out_shape = [jax.ShapeDtypeStruct((NPAD, 2 * H), jnp.float32)]
    return pl.pallas_call(
        body,
        grid=(NBLK,),
        in_specs=in_specs,
        out_specs=out_specs,
        out_shape=out_shape,
    )(*args)


def _tc_pool_fc(h2, batch3, Wfc, bfc):
    HW = h2.shape[1]
    C = Wfc.shape[1]

    def body(h_ref, b_ref, w_ref, bias_ref, o_ref, sums, cnt):
        i = pl.program_id(0)

        @pl.when(i == 0)
        def _():
            sums[...] = jnp.zeros_like(sums)
            cnt[...] = jnp.zeros_like(cnt)

        bvec = b_ref[0, 0, :]
        onehot = (lax.broadcasted_iota(jnp.int32, (G, BLK), 0)
                  == bvec[None, :]).astype(jnp.float32)
        sums[...] += jnp.dot(onehot, h_ref[...],
                             preferred_element_type=jnp.float32)
        cnt[...] += jnp.broadcast_to(
            jnp.sum(onehot, axis=1, keepdims=True), (G, 128))

        @pl.when(i == NBLK - 1)
        def _():
            pooled = sums[...] / jnp.maximum(cnt[:, 0:1], 1.0)
            logits = jnp.dot(pooled, w_ref[...],
                             preferred_element_type=jnp.float32) + bias_ref[...]
            m = jnp.max(logits, axis=1, keepdims=True)
            o_ref[...] = logits - m - jnp.log(
                jnp.sum(jnp.exp(logits - m), axis=1, keepdims=True))

    return pl.pallas_call(
        body,
        grid=(NBLK,),
        in_specs=[
            pl.BlockSpec((BLK, HW), lambda i: (i, 0)),
            pl.BlockSpec((1, 1, BLK), lambda i: (i, 0, 0)),
            pl.BlockSpec((HW, C), lambda i: (0, 0)),
            pl.BlockSpec((1, C), lambda i: (0, 0)),
        ],
        out_specs=pl.BlockSpec((G, C), lambda i: (0, 0)),
        out_shape=jax.ShapeDtypeStruct((G, C), jnp.float32),
        scratch_shapes=[
            pltpu.VMEM((G, HW), jnp.float32),
            pltpu.VMEM((G, 128), jnp.float32),
        ],
    )(h2, batch3, Wfc, bfc)


def kernel(x, edge_index, batch, W11, b11, W12, b12, W21, b21, W22, b22, Wfc, bfc):
    E = edge_index.shape[1]
    epad = ((E + NTILES * CHUNK - 1) // (NTILES * CHUNK)) * NTILES * CHUNK
    pad_idx = jnp.full((epad - E,), NPAD - 1, jnp.int32)
    srcp = jnp.concatenate([edge_index[0], pad_idx])
    dstp = jnp.concatenate([edge_index[1], pad_idx])
    xp = jnp.pad(x, ((0, NPAD - N), (0, 0)))
    batch3 = jnp.pad(batch, (0, NPAD - N), constant_values=G).reshape(NBLK, 1, BLK)

    W1f = W11.reshape(4 * F, 64)
    W1r = W12.reshape(4 * F, 64)
    W2f = W21.reshape(4 * F, 256)
    W2r = W22.reshape(4 * F, 256)
    b1f = b11.reshape(1, 64)
    b1r = b12.reshape(1, 64)
    b2f = b21.reshape(1, 256)
    b2r = b22.reshape(1, 256)
    bfc2 = bfc.reshape(1, -1)

    ones = jnp.ones((NPAD, F), jnp.float32)
    degr, degf = _sc_matvec(ones, ones, srcp, dstp)

    def cheb_layer(t0, Wf, Wr, bf, br, emit_z, z0f=None, z0r=None):
        if z0f is None:
            z0f, z0r = _tc_prep(t0, degf, degr)
        s1f, s1r = _sc_matvec(z0f, z0r, srcp, dstp)
        tx1f, tx1r, z1f, z1r = _tc_step(s1f, s1r, None, None, degf, degr, 1.0)
        s2f, s2r = _sc_matvec(z1f, z1r, srcp, dstp)
        tx2f, tx2r, z2f, z2r = _tc_step(s2f, s2r, t0, t0, degf, degr, 2.0)
        s3f, s3r = _sc_matvec(z2f, z2r, srcp, dstp)
        tx3f, tx3r, _, _ = _tc_step(s3f, s3r, tx1f, tx1r, degf, degr, 2.0)
        return _tc_layer_end((t0, tx1f, tx2f, tx3f), (t0, tx1r, tx2r, tx3r),
                             Wf, Wr, bf, br, degf, degr, emit_z)

    h, z0f2, z0r2 = cheb_layer(xp, W1f, W1r, b1f, b1r, True)
    (h2,) = cheb_layer(h, W2f, W2r, b2f, b2r, False, z0f2, z0r2)
    return _tc_pool_fc(h2, batch3, Wfc, bfc2)
